# baseline (speedup 1.0000x reference)
import jax
import jax.numpy as jnp
from jax import lax
from jax.experimental import pallas as pl
from jax.experimental.pallas import tpu as pltpu
from jax.experimental.pallas import tpu_sc as plsc

D_MODEL = 2048
NUM_MODES = 3
LANES = 16
CHUNKS = D_MODEL // LANES


def _row_gather_body(idx_hbm, table_hbm, out_hbm, idx_v, table_v, row_v,
                     sem_i, sem_t):
    cid = lax.axis_index("c")
    sid = lax.axis_index("s")

    @pl.when(jnp.logical_and(cid == 0, sid == 0))
    def _():
        cp_i = pltpu.make_async_copy(idx_hbm, idx_v, sem_i)
        cp_t = pltpu.make_async_copy(table_hbm, table_v, sem_t)
        cp_i.start()
        cp_t.start()
        cp_i.wait()
        cp_t.wait()
        m = idx_v[...]
        for j in range(CHUNKS):
            sl = pl.ds(j * LANES, LANES)
            v0 = table_v[0, sl]
            v1 = table_v[1, sl]
            v2 = table_v[2, sl]
            row_v[sl] = jnp.where(m == 0, v0, jnp.where(m == 1, v1, v2))
        pltpu.sync_copy(row_v, out_hbm)


def _mode_embed(idx, table):
    f = pl.kernel(
        _row_gather_body,
        out_type=jax.ShapeDtypeStruct((D_MODEL,), jnp.float32),
        mesh=plsc.VectorSubcoreMesh(core_axis_name="c", subcore_axis_name="s"),
        scratch_types=[
            pltpu.VMEM((LANES,), jnp.int32),
            pltpu.VMEM((NUM_MODES, D_MODEL), jnp.float32),
            pltpu.VMEM((D_MODEL,), jnp.float32),
            pltpu.SemaphoreType.DMA,
            pltpu.SemaphoreType.DMA,
        ],
    )
    return f(idx, table)


def kernel(mode, table):
    idx = jnp.broadcast_to(jnp.asarray(mode, jnp.int32), (LANES,))
    return _mode_embed(idx, table)


# speedup vs baseline: 1.2848x; 1.2848x over previous
import jax
import jax.numpy as jnp
from jax.experimental import pallas as pl
from jax.experimental.pallas import tpu as pltpu
from jax.experimental.pallas import tpu_sc as plsc

D_MODEL = 2048
NUM_MODES = 3


def _row_copy_body(idx_hbm, table_hbm, out_hbm, m_smem):
    pltpu.sync_copy(idx_hbm, m_smem)
    m = m_smem[0]
    pltpu.sync_copy(table_hbm.at[m], out_hbm)


def _mode_embed(idx, table):
    f = pl.kernel(
        _row_copy_body,
        out_type=jax.ShapeDtypeStruct((D_MODEL,), jnp.float32),
        mesh=plsc.ScalarSubcoreMesh(axis_name="c", num_cores=1),
        scratch_types=[
            pltpu.SMEM((1,), jnp.int32),
        ],
    )
    return f(idx, table)


def kernel(mode, table):
    idx = jnp.asarray(mode, jnp.int32).reshape(1)
    return _mode_embed(idx, table)


# speedup vs baseline: 10.9304x; 8.5075x over previous
import jax
import jax.numpy as jnp
from jax.experimental import pallas as pl
from jax.experimental.pallas import tpu as pltpu

D_MODEL = 2048
NUM_MODES = 3


def _tc_body(m_ref, table_ref, out_ref):
    m = m_ref[0]
    v = table_ref[...]
    out_ref[...] = jnp.where(m == 0, v[0:1],
                             jnp.where(m == 1, v[1:2], v[2:3]))


def _mode_embed_tc(idx, table):
    out = pl.pallas_call(
        _tc_body,
        in_specs=[
            pl.BlockSpec(memory_space=pltpu.SMEM),
            pl.BlockSpec(memory_space=pltpu.VMEM),
        ],
        out_specs=pl.BlockSpec(memory_space=pltpu.VMEM),
        out_shape=jax.ShapeDtypeStruct((1, D_MODEL), jnp.float32),
    )(idx, table)
    return out.reshape(D_MODEL)


def kernel(mode, table):
    idx = jnp.asarray(mode, jnp.int32).reshape(1)
    return _mode_embed_tc(idx, table)
